# Initial kernel scaffold; baseline (speedup 1.0000x reference)
import jax
import jax.numpy as jnp
from jax.experimental import pallas as pl


def kernel(x, edge_index, batch, params):
    raise NotImplementedError("write your pallas kernel here")


# speedup vs baseline: 4.9225x; 4.9225x over previous
import functools

import jax
import jax.numpy as jnp
from jax import lax
from jax.experimental import pallas as pl
from jax.experimental.pallas import tpu as pltpu
from jax.experimental.pallas import tpu_sc as plsc

_N = 50000
_E = 800000
_H = 64
_G = 128

_BS = 512
_NB = 98
_NP = _BS * _NB
_CH = 128
_NCH = 392
_EP = 16 * _NCH * _CH
_RPT = _NP // 16
_ZB = 224
_ZN = _RPT // _ZB
_QW = 8


def _sc_mesh():
    return plsc.VectorSubcoreMesh(core_axis_name="c", subcore_axis_name="s",
                                  num_cores=2, num_subcores=16)


def _make_sc_quarter(k):
    @functools.partial(
        pl.kernel,
        out_type=jax.ShapeDtypeStruct((_NP, 8, _QW), jnp.float32),
        mesh=_sc_mesh(),
        scratch_types=[
            pltpu.VMEM((_NCH, _CH), jnp.int32),
            pltpu.VMEM((_NCH, _CH), jnp.int32),
            pltpu.VMEM((_CH, _QW), jnp.float32),
            pltpu.VMEM((_ZB, _QW), jnp.float32),
            pltpu.VMEM((_ZB, _QW), jnp.float32),
            pltpu.VMEM_SHARED((_NP, _QW), jnp.float32),
            pltpu.SemaphoreType.DMA,
        ],
        compiler_params=pltpu.CompilerParams(use_tc_tiling_on_sc=False),
    )
    def sc_q(tbl, src_all, dst_r, zrows,
             out, src_v, dst_v, rows, zbuf, wbuf, acc, sem):
        c = lax.axis_index("c")
        s = lax.axis_index("s")
        base = s * _RPT

        pltpu.sync_copy(zrows, zbuf)
        pltpu.sync_copy(src_all.at[c, s], src_v)
        pltpu.sync_copy(dst_r.at[s], dst_v)

        def zero_step(i, carry):
            pltpu.sync_copy(zbuf, acc.at[pl.ds(base + i * _ZB, _ZB)])
            return carry
        lax.fori_loop(0, _ZN, zero_step, 0)
        plsc.subcore_barrier()

        def chunk(i, carry):
            pltpu.async_copy(tbl.at[src_v.at[i]], rows, sem).wait()
            pltpu.sync_copy(rows, acc.at[dst_v.at[i]], add=True)
            return carry
        lax.fori_loop(0, _NCH, chunk, 0)
        plsc.subcore_barrier()

        q = 2 * k + c

        def w_step(i, carry):
            pltpu.sync_copy(acc.at[pl.ds(base + i * _ZB, _ZB)], wbuf)
            pltpu.sync_copy(wbuf, out.at[pl.ds(base + i * _ZB, _ZB), q])
            return carry
        lax.fori_loop(0, _ZN, w_step, 0)

    return sc_q


def _make_sc_deg():
    out_t = (jax.ShapeDtypeStruct((_NP,), jnp.float32),
             jax.ShapeDtypeStruct((_NP,), jnp.float32))

    @functools.partial(
        pl.kernel,
        out_type=out_t,
        mesh=_sc_mesh(),
        scratch_types=[
            pltpu.VMEM((_NCH, _CH), jnp.int32),
            pltpu.VMEM((_CH,), jnp.float32),
            pltpu.VMEM((_ZB,), jnp.float32),
            pltpu.VMEM_SHARED((_NP,), jnp.float32),
            pltpu.SemaphoreType.DMA,
        ],
        compiler_params=pltpu.CompilerParams(use_tc_tiling_on_sc=False),
    )
    def sc_deg(ones_tbl, dst_r, zrows, out_a, out_b,
               dst_v, ones_v, zbuf, acc, sem):
        c = lax.axis_index("c")
        s = lax.axis_index("s")
        base = s * _RPT

        pltpu.sync_copy(zrows, zbuf)
        pltpu.sync_copy(ones_tbl.at[pl.ds(0, _CH)], ones_v)
        pltpu.sync_copy(dst_r.at[s], dst_v)

        def zero_step(i, carry):
            pltpu.sync_copy(zbuf, acc.at[pl.ds(base + i * _ZB, _ZB)])
            return carry
        lax.fori_loop(0, _ZN, zero_step, 0)
        plsc.subcore_barrier()

        def chunk(i, carry):
            pltpu.sync_copy(ones_v, acc.at[dst_v.at[2 * i + c]], add=True)
            return carry
        lax.fori_loop(0, _NCH // 2, chunk, 0)
        plsc.subcore_barrier()

        def writeout(out):
            def w_step(i, carry):
                pltpu.sync_copy(acc.at[pl.ds(base + i * _ZB, _ZB)], zbuf)
                pltpu.sync_copy(zbuf, out.at[pl.ds(base + i * _ZB, _ZB)])
                return carry
            lax.fori_loop(0, _ZN, w_step, 0)

        @pl.when(c == 0)
        def _():
            writeout(out_a)

        @pl.when(c == 1)
        def _():
            writeout(out_b)

    return sc_deg


@functools.lru_cache(maxsize=None)
def _sc_quarter(k):
    return _make_sc_quarter(k)


@functools.lru_cache(maxsize=None)
def _sc_deg_k():
    return _make_sc_deg()


def _full(a):
    return pl.BlockSpec(a.shape, lambda i: (0,) * a.ndim)


def _rows(w):
    return pl.BlockSpec((_BS, w), lambda i: (i, 0))


def _b1_body(x_ref, dega_ref, degb_ref, w1_ref, hs_ref, dinv_ref):
    d = lax.rsqrt(dega_ref[...] + degb_ref[...] + 1.0)
    hw = jnp.dot(x_ref[...], w1_ref[...], preferred_element_type=jnp.float32)
    hs_ref[...] = hw * d
    dinv_ref[...] = d


def _call_b1(x_p, dega, degb, w1):
    return pl.pallas_call(
        _b1_body,
        grid=(_NB,),
        in_specs=[_rows(3), _rows(1), _rows(1), _full(w1)],
        out_specs=[_rows(_H), _rows(1)],
        out_shape=[
            jax.ShapeDtypeStruct((_NP, _H), jnp.float32),
            jax.ShapeDtypeStruct((_NP, 1), jnp.float32),
        ],
    )(x_p, dega, degb, w1)


def _d_body(agg0_ref, agg1_ref, agg2_ref, agg3_ref, hs_ref, dinv_ref,
            b_ref, y_ref, st_ref, acc):
    i = pl.program_id(0)

    @pl.when(i == 0)
    def _():
        acc[...] = jnp.zeros_like(acc)

    agg = jnp.concatenate(
        [agg0_ref[...][:, 0:16], agg1_ref[...][:, 16:32],
         agg2_ref[...][:, 32:48], agg3_ref[...][:, 48:64]], axis=1)
    y = (agg + hs_ref[...]) * dinv_ref[...] + b_ref[...]
    y_ref[...] = y

    row = i * _BS + lax.broadcasted_iota(jnp.int32, (_BS, 1), 0)
    ym = jnp.where(row < _N, y, 0.0)
    s = jnp.sum(ym, axis=0, keepdims=True)
    ss = jnp.sum(ym * ym, axis=0, keepdims=True)
    acc[...] = acc[...] + jnp.concatenate([s, ss], axis=0)

    @pl.when(i == _NB - 1)
    def _():
        st_ref[...] = acc[...]


def _call_d(aggs, hs, dinv, b):
    return pl.pallas_call(
        _d_body,
        grid=(_NB,),
        in_specs=[_rows(_H)] * 5 + [_rows(1), _full(b)],
        out_specs=[_rows(_H), pl.BlockSpec((2, _H), lambda i: (0, 0))],
        out_shape=[
            jax.ShapeDtypeStruct((_NP, _H), jnp.float32),
            jax.ShapeDtypeStruct((2, _H), jnp.float32),
        ],
        scratch_shapes=[pltpu.VMEM((2, _H), jnp.float32)],
        compiler_params=pltpu.CompilerParams(
            dimension_semantics=("arbitrary",)),
    )(*aggs, hs, dinv, b)


def _bn_relu(y, st, g, be):
    m = st[0:1, :] * (1.0 / _N)
    v = st[1:2, :] * (1.0 / _N) - m * m
    return jnp.maximum((y - m) * lax.rsqrt(v + 1e-5) * g + be, 0.0)


def _e_body(y_ref, st_ref, g_ref, be_ref, w_ref, dinv_ref, hs_ref):
    h = _bn_relu(y_ref[...], st_ref[...], g_ref[...], be_ref[...])
    hw = jnp.dot(h, w_ref[...], preferred_element_type=jnp.float32)
    hs_ref[...] = hw * dinv_ref[...]


def _call_e(y, st, g, be, w, dinv):
    return pl.pallas_call(
        _e_body,
        grid=(_NB,),
        in_specs=[_rows(_H), _full(st), _full(g), _full(be), _full(w),
                  _rows(1)],
        out_specs=_rows(_H),
        out_shape=jax.ShapeDtypeStruct((_NP, _H), jnp.float32),
    )(y, st, g, be, w, dinv)


def _p_body(h_ref, bat_ref, l1w_ref, l1b_ref, gf1_ref, bf1_ref,
            l2w_ref, l2b_ref, gf2_ref, bf2_ref, l3w_ref, l3b_ref,
            out_ref, sum_acc, max_acc, cnt_acc):
    i = pl.program_id(0)

    @pl.when(i == 0)
    def _():
        sum_acc[...] = jnp.zeros_like(sum_acc)
        cnt_acc[...] = jnp.zeros_like(cnt_acc)
        max_acc[...] = jnp.full_like(max_acc, -jnp.inf)

    bat = bat_ref[...]
    h = h_ref[...]
    lo = jnp.min(bat)
    hi = jnp.minimum(jnp.max(bat), _G - 1)
    gid = lax.broadcasted_iota(jnp.int32, (_G, 1), 0)

    def seg(g, carry):
        m = bat == g
        s = jnp.sum(jnp.where(m, h, 0.0), axis=0, keepdims=True)
        c = jnp.sum(jnp.where(m, 1.0, 0.0))
        mx = jnp.max(jnp.where(m, h, -jnp.inf), axis=0, keepdims=True)
        oh = gid == g
        ohf = jnp.where(oh, 1.0, 0.0)
        sum_acc[...] = sum_acc[...] + ohf * s
        cnt_acc[...] = cnt_acc[...] + ohf * c
        max_acc[...] = jnp.where(oh, jnp.maximum(max_acc[...], mx),
                                 max_acc[...])
        return carry

    lax.fori_loop(lo, hi + 1, seg, 0)

    @pl.when(i == _NB - 1)
    def _():
        mean = sum_acc[...] / jnp.maximum(cnt_acc[...], 1.0)
        z = jnp.concatenate([mean, max_acc[...]], axis=1)

        def bn_g(t, g, b):
            m = jnp.mean(t, axis=0, keepdims=True)
            v = jnp.mean(t * t, axis=0, keepdims=True) - m * m
            return (t - m) * lax.rsqrt(v + 1e-5) * g + b

        z = jnp.dot(z, l1w_ref[...], preferred_element_type=jnp.float32)
        z = jnp.maximum(bn_g(z + l1b_ref[...], gf1_ref[...], bf1_ref[...]),
                        0.0)
        z = jnp.dot(z, l2w_ref[...], preferred_element_type=jnp.float32)
        z = jnp.maximum(bn_g(z + l2b_ref[...], gf2_ref[...], bf2_ref[...]),
                        0.0)
        out_ref[...] = (jnp.dot(z, l3w_ref[...],
                                preferred_element_type=jnp.float32)
                        + l3b_ref[...])


def _call_p(h, bat_p, p):
    args = [h, bat_p,
            p["lin1_W"], p["lin1_b"].reshape(1, _H),
            p["gf1"].reshape(1, _H), p["bf1"].reshape(1, _H),
            p["lin2_W"], p["lin2_b"].reshape(1, _H // 2),
            p["gf2"].reshape(1, _H // 2), p["bf2"].reshape(1, _H // 2),
            p["lin3_W"], p["lin3_b"].reshape(1, 2)]
    return pl.pallas_call(
        _p_body,
        grid=(_NB,),
        in_specs=[_rows(_H), _rows(1)] + [_full(a) for a in args[2:]],
        out_specs=pl.BlockSpec((_G, 2), lambda i: (0, 0)),
        out_shape=jax.ShapeDtypeStruct((_G, 2), jnp.float32),
        scratch_shapes=[pltpu.VMEM((_G, _H), jnp.float32),
                        pltpu.VMEM((_G, _H), jnp.float32),
                        pltpu.VMEM((_G, 1), jnp.float32)],
        compiler_params=pltpu.CompilerParams(
            dimension_semantics=("arbitrary",)),
    )(*args)


def kernel(x, edge_index, batch, params):
    p = params
    src = edge_index[0].astype(jnp.int32)
    dst = edge_index[1].astype(jnp.int32)

    src8 = jnp.concatenate(
        [src * 8, jnp.zeros((_EP - _E,), jnp.int32)]).reshape(16, _NCH, _CH)
    dst_r = jnp.concatenate(
        [dst, jnp.full((_EP - _E,), _N, jnp.int32)]).reshape(16, _NCH, _CH)
    sas = [jnp.stack([src8 + 2 * k, src8 + 2 * k + 1]) for k in range(4)]

    x_p = jnp.concatenate(
        [x, jnp.zeros((_NP - _N, 3), jnp.float32)], axis=0)
    bat_p = jnp.concatenate(
        [batch.astype(jnp.int32), jnp.full((_NP - _N,), _G, jnp.int32)]
    ).reshape(_NP, 1)

    ones_tbl = jnp.ones((_NP,), jnp.float32)
    zeros1 = jnp.zeros((_ZB,), jnp.float32)
    zeros2 = jnp.zeros((_ZB, _QW), jnp.float32)

    dega, degb = _sc_deg_k()(ones_tbl, dst_r, zeros1)
    hs, dinv = _call_b1(x_p, dega.reshape(_NP, 1),
                        degb.reshape(_NP, 1), p["W1"])

    wn = jnp.stack([p["W2"], p["W3"], jnp.eye(_H, dtype=jnp.float32)])
    bn = jnp.stack([p["b1"].reshape(1, _H), p["b2"].reshape(1, _H),
                    p["b3"].reshape(1, _H)])
    gn = jnp.stack([p["g1"].reshape(1, _H), p["g2"].reshape(1, _H),
                    p["g3"].reshape(1, _H)])
    ben = jnp.stack([p["be1"].reshape(1, _H), p["be2"].reshape(1, _H),
                     p["be3"].reshape(1, _H)])
    dsc = jnp.stack([dinv, dinv, jnp.ones((_NP, 1), jnp.float32)])

    def layer(l, hs):
        tbl = hs.reshape(8 * _NP, _QW)
        aggs = [_sc_quarter(k)(tbl, sas[k], dst_r, zeros2).reshape(_NP, _H)
                for k in range(4)]
        y, st = _call_d(aggs, hs, dinv,
                        lax.dynamic_index_in_dim(bn, l, keepdims=False))
        return _call_e(
            y, st,
            lax.dynamic_index_in_dim(gn, l, keepdims=False),
            lax.dynamic_index_in_dim(ben, l, keepdims=False),
            lax.dynamic_index_in_dim(wn, l, keepdims=False),
            lax.dynamic_index_in_dim(dsc, l, keepdims=False))

    nlayers = 3 + jnp.minimum(src[0], 0)
    hs = lax.fori_loop(0, nlayers, layer, hs)
    return _call_p(hs, bat_p, p)


# speedup vs baseline: 6.4504x; 1.3104x over previous
import functools

import jax
import jax.numpy as jnp
from jax import lax
from jax.experimental import pallas as pl
from jax.experimental.pallas import tpu as pltpu
from jax.experimental.pallas import tpu_sc as plsc

_N = 50000
_E = 800000
_H = 64
_G = 128

_BS = 512
_NB = 98
_NP = _BS * _NB
_CH = 128
_NCH = 392
_EP = 16 * _NCH * _CH
_RPT = _NP // 16
_ZB = 224
_ZN = _RPT // _ZB
_QW = 8
_NCHS = _NCH + 2
_ARF = 50048
_RPTF = _ARF // 16
_ZBF = 136
_ZNF = _RPTF // _ZBF


def _sc_mesh():
    return plsc.VectorSubcoreMesh(core_axis_name="c", subcore_axis_name="s",
                                  num_cores=2, num_subcores=16)


def _make_sc_quarter(k):
    @functools.partial(
        pl.kernel,
        out_type=jax.ShapeDtypeStruct((_NP, 8, _QW), jnp.float32),
        mesh=_sc_mesh(),
        scratch_types=[
            pltpu.VMEM((_NCHS, _CH), jnp.int32),
            pltpu.VMEM((_NCH, _CH), jnp.int32),
            pltpu.VMEM((_CH, _QW), jnp.float32),
            pltpu.VMEM((_CH, _QW), jnp.float32),
            pltpu.VMEM((_ZBF, _QW), jnp.float32),
            pltpu.VMEM((_ZBF, _QW), jnp.float32),
            pltpu.VMEM_SHARED((_ARF, _QW), jnp.float32),
            pltpu.SemaphoreType.DMA,
            pltpu.SemaphoreType.DMA,
        ],
        compiler_params=pltpu.CompilerParams(use_tc_tiling_on_sc=False),
    )
    def sc_q(tbl, src_all, dst_r, zrows,
             out, src_v, dst_v, rows0, rows1, zbuf, wbuf, acc, sem0, sem1):
        c = lax.axis_index("c")
        s = lax.axis_index("s")
        base = s * _RPTF

        pltpu.sync_copy(zrows, zbuf)
        pltpu.sync_copy(src_all.at[c, s], src_v)
        pltpu.sync_copy(dst_r.at[s], dst_v)

        def zero_step(i, carry):
            pltpu.sync_copy(zbuf, acc.at[pl.ds(base + i * _ZBF, _ZBF)])
            return carry
        lax.fori_loop(0, _ZNF, zero_step, 0)
        plsc.subcore_barrier()

        pltpu.async_copy(tbl.at[src_v.at[0]], rows0, sem0)

        def chunk(i, carry):
            j = 2 * i
            pltpu.make_async_copy(tbl.at[src_v.at[j]], rows0, sem0).wait()
            pltpu.async_copy(tbl.at[src_v.at[j + 1]], rows1, sem1)
            pltpu.sync_copy(rows0, acc.at[dst_v.at[j]], add=True)
            pltpu.async_copy(tbl.at[src_v.at[j + 2]], rows0, sem0)
            pltpu.make_async_copy(tbl.at[src_v.at[j + 1]], rows1, sem1).wait()
            pltpu.sync_copy(rows1, acc.at[dst_v.at[j + 1]], add=True)
            return carry
        lax.fori_loop(0, _NCH // 2, chunk, 0)
        pltpu.make_async_copy(tbl.at[src_v.at[_NCH]], rows0, sem0).wait()
        plsc.subcore_barrier()

        q = 2 * k + c

        def w_step(i, carry):
            pltpu.sync_copy(acc.at[pl.ds(base + i * _ZBF, _ZBF)], wbuf)
            pltpu.sync_copy(wbuf, out.at[pl.ds(base + i * _ZBF, _ZBF), q])
            return carry
        lax.fori_loop(0, _ZNF, w_step, 0)

    return sc_q


def _make_sc_deg():
    out_t = (jax.ShapeDtypeStruct((_NP,), jnp.float32),
             jax.ShapeDtypeStruct((_NP,), jnp.float32))

    @functools.partial(
        pl.kernel,
        out_type=out_t,
        mesh=_sc_mesh(),
        scratch_types=[
            pltpu.VMEM((_NCH, _CH), jnp.int32),
            pltpu.VMEM((_CH,), jnp.float32),
            pltpu.VMEM((_ZB,), jnp.float32),
            pltpu.VMEM_SHARED((_NP,), jnp.float32),
            pltpu.SemaphoreType.DMA,
        ],
        compiler_params=pltpu.CompilerParams(use_tc_tiling_on_sc=False),
    )
    def sc_deg(ones_tbl, dst_r, zrows, out_a, out_b,
               dst_v, ones_v, zbuf, acc, sem):
        c = lax.axis_index("c")
        s = lax.axis_index("s")
        base = s * _RPT

        pltpu.sync_copy(zrows, zbuf)
        pltpu.sync_copy(ones_tbl.at[pl.ds(0, _CH)], ones_v)
        pltpu.sync_copy(dst_r.at[s], dst_v)

        def zero_step(i, carry):
            pltpu.sync_copy(zbuf, acc.at[pl.ds(base + i * _ZB, _ZB)])
            return carry
        lax.fori_loop(0, _ZN, zero_step, 0)
        plsc.subcore_barrier()

        def chunk(i, carry):
            pltpu.sync_copy(ones_v, acc.at[dst_v.at[2 * i + c]], add=True)
            return carry
        lax.fori_loop(0, _NCH // 2, chunk, 0)
        plsc.subcore_barrier()

        def writeout(out):
            def w_step(i, carry):
                pltpu.sync_copy(acc.at[pl.ds(base + i * _ZB, _ZB)], zbuf)
                pltpu.sync_copy(zbuf, out.at[pl.ds(base + i * _ZB, _ZB)])
                return carry
            lax.fori_loop(0, _ZN, w_step, 0)

        @pl.when(c == 0)
        def _():
            writeout(out_a)

        @pl.when(c == 1)
        def _():
            writeout(out_b)

    return sc_deg


@functools.lru_cache(maxsize=None)
def _sc_quarter(k):
    return _make_sc_quarter(k)


@functools.lru_cache(maxsize=None)
def _sc_deg_k():
    return _make_sc_deg()


def _full(a):
    return pl.BlockSpec(a.shape, lambda i: (0,) * a.ndim)


def _rows(w):
    return pl.BlockSpec((_BS, w), lambda i: (i, 0))


def _b1_body(x_ref, dega_ref, degb_ref, w1_ref, hs_ref, dinv_ref):
    d = lax.rsqrt(dega_ref[...] + degb_ref[...] + 1.0)
    hw = jnp.dot(x_ref[...], w1_ref[...], preferred_element_type=jnp.float32)
    hs_ref[...] = hw * d
    dinv_ref[...] = d


def _call_b1(x_p, dega, degb, w1):
    return pl.pallas_call(
        _b1_body,
        grid=(_NB,),
        in_specs=[_rows(3), _rows(1), _rows(1), _full(w1)],
        out_specs=[_rows(_H), _rows(1)],
        out_shape=[
            jax.ShapeDtypeStruct((_NP, _H), jnp.float32),
            jax.ShapeDtypeStruct((_NP, 1), jnp.float32),
        ],
    )(x_p, dega, degb, w1)


def _d_body(agg0_ref, agg1_ref, agg2_ref, agg3_ref, hs_ref, dinv_ref,
            b_ref, y_ref, st_ref, acc):
    i = pl.program_id(0)

    @pl.when(i == 0)
    def _():
        acc[...] = jnp.zeros_like(acc)

    agg = jnp.concatenate(
        [agg0_ref[...][:, 0:16], agg1_ref[...][:, 16:32],
         agg2_ref[...][:, 32:48], agg3_ref[...][:, 48:64]], axis=1)
    y = (agg + hs_ref[...]) * dinv_ref[...] + b_ref[...]
    y_ref[...] = y

    row = i * _BS + lax.broadcasted_iota(jnp.int32, (_BS, 1), 0)
    ym = jnp.where(row < _N, y, 0.0)
    s = jnp.sum(ym, axis=0, keepdims=True)
    ss = jnp.sum(ym * ym, axis=0, keepdims=True)
    acc[...] = acc[...] + jnp.concatenate([s, ss], axis=0)

    @pl.when(i == _NB - 1)
    def _():
        st_ref[...] = acc[...]


def _call_d(aggs, hs, dinv, b):
    return pl.pallas_call(
        _d_body,
        grid=(_NB,),
        in_specs=[_rows(_H)] * 5 + [_rows(1), _full(b)],
        out_specs=[_rows(_H), pl.BlockSpec((2, _H), lambda i: (0, 0))],
        out_shape=[
            jax.ShapeDtypeStruct((_NP, _H), jnp.float32),
            jax.ShapeDtypeStruct((2, _H), jnp.float32),
        ],
        scratch_shapes=[pltpu.VMEM((2, _H), jnp.float32)],
        compiler_params=pltpu.CompilerParams(
            dimension_semantics=("arbitrary",)),
    )(*aggs, hs, dinv, b)


def _bn_relu(y, st, g, be):
    m = st[0:1, :] * (1.0 / _N)
    v = st[1:2, :] * (1.0 / _N) - m * m
    return jnp.maximum((y - m) * lax.rsqrt(v + 1e-5) * g + be, 0.0)


def _e_body(y_ref, st_ref, g_ref, be_ref, w_ref, dinv_ref, hs_ref):
    h = _bn_relu(y_ref[...], st_ref[...], g_ref[...], be_ref[...])
    hw = jnp.dot(h, w_ref[...], preferred_element_type=jnp.float32)
    hs_ref[...] = hw * dinv_ref[...]


def _call_e(y, st, g, be, w, dinv):
    return pl.pallas_call(
        _e_body,
        grid=(_NB,),
        in_specs=[_rows(_H), _full(st), _full(g), _full(be), _full(w),
                  _rows(1)],
        out_specs=_rows(_H),
        out_shape=jax.ShapeDtypeStruct((_NP, _H), jnp.float32),
    )(y, st, g, be, w, dinv)


def _p_body(h_ref, bat_ref, l1w_ref, l1b_ref, gf1_ref, bf1_ref,
            l2w_ref, l2b_ref, gf2_ref, bf2_ref, l3w_ref, l3b_ref,
            out_ref, sum_acc, max_acc, cnt_acc):
    i = pl.program_id(0)

    @pl.when(i == 0)
    def _():
        sum_acc[...] = jnp.zeros_like(sum_acc)
        cnt_acc[...] = jnp.zeros_like(cnt_acc)
        max_acc[...] = jnp.full_like(max_acc, -jnp.inf)

    bat = bat_ref[...]
    h = h_ref[...]
    lo = jnp.min(bat)
    hi = jnp.minimum(jnp.max(bat), _G - 1)
    gid = lax.broadcasted_iota(jnp.int32, (_G, 1), 0)

    def seg(g, carry):
        m = bat == g
        s = jnp.sum(jnp.where(m, h, 0.0), axis=0, keepdims=True)
        c = jnp.sum(jnp.where(m, 1.0, 0.0))
        mx = jnp.max(jnp.where(m, h, -jnp.inf), axis=0, keepdims=True)
        oh = gid == g
        ohf = jnp.where(oh, 1.0, 0.0)
        sum_acc[...] = sum_acc[...] + ohf * s
        cnt_acc[...] = cnt_acc[...] + ohf * c
        max_acc[...] = jnp.where(oh, jnp.maximum(max_acc[...], mx),
                                 max_acc[...])
        return carry

    lax.fori_loop(lo, hi + 1, seg, 0)

    @pl.when(i == _NB - 1)
    def _():
        mean = sum_acc[...] / jnp.maximum(cnt_acc[...], 1.0)
        z = jnp.concatenate([mean, max_acc[...]], axis=1)

        def bn_g(t, g, b):
            m = jnp.mean(t, axis=0, keepdims=True)
            v = jnp.mean(t * t, axis=0, keepdims=True) - m * m
            return (t - m) * lax.rsqrt(v + 1e-5) * g + b

        z = jnp.dot(z, l1w_ref[...], preferred_element_type=jnp.float32)
        z = jnp.maximum(bn_g(z + l1b_ref[...], gf1_ref[...], bf1_ref[...]),
                        0.0)
        z = jnp.dot(z, l2w_ref[...], preferred_element_type=jnp.float32)
        z = jnp.maximum(bn_g(z + l2b_ref[...], gf2_ref[...], bf2_ref[...]),
                        0.0)
        out_ref[...] = (jnp.dot(z, l3w_ref[...],
                                preferred_element_type=jnp.float32)
                        + l3b_ref[...])


def _call_p(h, bat_p, p):
    args = [h, bat_p,
            p["lin1_W"], p["lin1_b"].reshape(1, _H),
            p["gf1"].reshape(1, _H), p["bf1"].reshape(1, _H),
            p["lin2_W"], p["lin2_b"].reshape(1, _H // 2),
            p["gf2"].reshape(1, _H // 2), p["bf2"].reshape(1, _H // 2),
            p["lin3_W"], p["lin3_b"].reshape(1, 2)]
    return pl.pallas_call(
        _p_body,
        grid=(_NB,),
        in_specs=[_rows(_H), _rows(1)] + [_full(a) for a in args[2:]],
        out_specs=pl.BlockSpec((_G, 2), lambda i: (0, 0)),
        out_shape=jax.ShapeDtypeStruct((_G, 2), jnp.float32),
        scratch_shapes=[pltpu.VMEM((_G, _H), jnp.float32),
                        pltpu.VMEM((_G, _H), jnp.float32),
                        pltpu.VMEM((_G, 1), jnp.float32)],
        compiler_params=pltpu.CompilerParams(
            dimension_semantics=("arbitrary",)),
    )(*args)


def kernel(x, edge_index, batch, params):
    p = params
    src = edge_index[0].astype(jnp.int32)
    dst = edge_index[1].astype(jnp.int32)

    src8 = jnp.concatenate(
        [src * 8, jnp.zeros((_EP - _E,), jnp.int32)]).reshape(16, _NCH, _CH)
    src8 = jnp.concatenate(
        [src8, jnp.zeros((16, 2, _CH), jnp.int32)], axis=1)
    dst_r = jnp.concatenate(
        [dst, jnp.full((_EP - _E,), _N, jnp.int32)]).reshape(16, _NCH, _CH)
    sas = [jnp.stack([src8 + 2 * k, src8 + 2 * k + 1]) for k in range(4)]

    x_p = jnp.concatenate(
        [x, jnp.zeros((_NP - _N, 3), jnp.float32)], axis=0)
    bat_p = jnp.concatenate(
        [batch.astype(jnp.int32), jnp.full((_NP - _N,), _G, jnp.int32)]
    ).reshape(_NP, 1)

    ones_tbl = jnp.ones((_NP,), jnp.float32)
    zeros1 = jnp.zeros((_ZB,), jnp.float32)
    zeros2 = jnp.zeros((_ZBF, _QW), jnp.float32)

    dega, degb = _sc_deg_k()(ones_tbl, dst_r, zeros1)
    hs, dinv = _call_b1(x_p, dega.reshape(_NP, 1),
                        degb.reshape(_NP, 1), p["W1"])

    wn = jnp.stack([p["W2"], p["W3"], jnp.eye(_H, dtype=jnp.float32)])
    bn = jnp.stack([p["b1"].reshape(1, _H), p["b2"].reshape(1, _H),
                    p["b3"].reshape(1, _H)])
    gn = jnp.stack([p["g1"].reshape(1, _H), p["g2"].reshape(1, _H),
                    p["g3"].reshape(1, _H)])
    ben = jnp.stack([p["be1"].reshape(1, _H), p["be2"].reshape(1, _H),
                     p["be3"].reshape(1, _H)])
    dsc = jnp.stack([dinv, dinv, jnp.ones((_NP, 1), jnp.float32)])

    def layer(l, hs):
        tbl = hs.reshape(8 * _NP, _QW)
        aggs = [_sc_quarter(k)(tbl, sas[k], dst_r, zeros2).reshape(_NP, _H)
                for k in range(4)]
        y, st = _call_d(aggs, hs, dinv,
                        lax.dynamic_index_in_dim(bn, l, keepdims=False))
        return _call_e(
            y, st,
            lax.dynamic_index_in_dim(gn, l, keepdims=False),
            lax.dynamic_index_in_dim(ben, l, keepdims=False),
            lax.dynamic_index_in_dim(wn, l, keepdims=False),
            lax.dynamic_index_in_dim(dsc, l, keepdims=False))

    nlayers = 3 + jnp.minimum(src[0], 0)
    hs = lax.fori_loop(0, nlayers, layer, hs)
    return _call_p(hs, bat_p, p)
